# baseline (speedup 1.0000x reference)
import functools

import jax
import jax.numpy as jnp
import numpy as np
from jax import lax
from jax.experimental import pallas as pl
from jax.experimental.pallas import tpu as pltpu
from jax.experimental.pallas import tpu_sc as plsc

_N = 4194304
_TBL_WORDS = 2097152
_NC, _NS = 2, 16
_NW = _NC * _NS
_PPW = _N // _NW
_C = 4096
_NCHUNK = _PPW // _C
_R = _C // 128
_GPR = 8

_F_HALF = np.float32(0.5)
_F_ONE = np.float32(1.0)
_F_RES = np.float32(128.0)
_F_TOP = np.float32(1.0 - 1e-5)
_SIGN = np.int32(-2147483648)


def _sc_body(pos_hbm, tbl_hbm, out_hbm, posv, widxv, vidxv, wordsv, outv, sem):
    wid = lax.axis_index("s") * _NC + lax.axis_index("c")
    base = wid * _PPW
    lane3 = lax.iota(jnp.int32, 16) * 3

    def chunk(c, _):
        start = base + c * _C
        pltpu.sync_copy(pos_hbm.at[pl.ds(start * 3, _C * 3)], posv)

        def row_compute(j, _):
            for gg in range(_GPR):
                g = j * _GPR + gg
                ix = lane3 + g * 48
                px = plsc.load_gather(posv, [ix])
                py = plsc.load_gather(posv, [ix + 1])
                pz = plsc.load_gather(posv, [ix + 2])
                tx = (px + _F_ONE) * _F_HALF - _F_HALF
                ty = (py + _F_ONE) * _F_HALF - _F_HALF
                tz = (pz + _F_ONE) * _F_HALF - _F_HALF
                m = jnp.maximum(jnp.maximum(jnp.abs(tx), jnp.abs(ty)),
                                jnp.abs(tz))
                ebits = plsc.bitcast(m, jnp.int32) >> 23
                mip_raw = jnp.where(m > 0.0, ebits - 125, 1)
                valid = mip_raw < 4
                mip = jnp.minimum(jnp.maximum(mip_raw, 0), 3)
                scale = plsc.bitcast((127 - mip) << 23, jnp.float32)
                vx = jnp.minimum(jnp.maximum(tx * scale + _F_HALF, 0.0), _F_TOP)
                vy = jnp.minimum(jnp.maximum(ty * scale + _F_HALF, 0.0), _F_TOP)
                vz = jnp.minimum(jnp.maximum(tz * scale + _F_HALF, 0.0), _F_TOP)
                xi = (vx * _F_RES).astype(jnp.int32)
                yi = (vy * _F_RES).astype(jnp.int32)
                zi = (vz * _F_RES).astype(jnp.int32)
                idx = xi * 16384 + yi * 128 + zi + (mip << 21)
                widxv[pl.ds(g * 16, 16)] = idx >> 2
                vidxv[pl.ds(g * 16, 16)] = jnp.where(valid, idx, idx | _SIGN)
            pltpu.async_copy(
                tbl_hbm.at[widxv.at[pl.ds(j * 128, 128)]],
                wordsv.at[pl.ds(j * 128, 128)], sem)
            return ()

        lax.fori_loop(0, _R, row_compute, (), unroll=False)

        def row_drain(j, _):
            pltpu.make_async_copy(
                tbl_hbm.at[widxv.at[pl.ds(j * 128, 128)]],
                wordsv.at[pl.ds(j * 128, 128)], sem).wait()
            return ()

        lax.fori_loop(0, _R, row_drain, (), unroll=False)

        def row_extract(j, _):
            for gg in range(_GPR):
                g = j * _GPR + gg
                w = wordsv[pl.ds(g * 16, 16)]
                vidx = vidxv[pl.ds(g * 16, 16)]
                bit = (w >> ((vidx & 3) << 3)) & 1
                outv[pl.ds(g * 16, 16)] = jnp.where(vidx >= 0, bit, 0)
            return ()

        lax.fori_loop(0, _R, row_extract, (), unroll=False)
        pltpu.sync_copy(outv, out_hbm.at[pl.ds(start, _C)])
        return ()

    lax.fori_loop(0, _NCHUNK, chunk, (), unroll=False)


@jax.jit
def _occupied(posf, tbl):
    mesh = plsc.VectorSubcoreMesh(
        core_axis_name="c", subcore_axis_name="s",
        num_cores=_NC, num_subcores=_NS)
    f = pl.kernel(
        _sc_body,
        out_type=jax.ShapeDtypeStruct((_N,), jnp.int32),
        mesh=mesh,
        compiler_params=pltpu.CompilerParams(needs_layout_passes=False),
        scratch_types=[
            pltpu.VMEM((3 * _C,), jnp.float32),
            pltpu.VMEM((_C,), jnp.int32),
            pltpu.VMEM((_C,), jnp.int32),
            pltpu.VMEM((_C,), jnp.int32),
            pltpu.VMEM((_C,), jnp.int32),
            pltpu.SemaphoreType.DMA,
        ],
    )
    return f(posf, tbl)


def kernel(pos, occs_binary, aabbs):
    posf = pos.reshape(-1)
    occ8 = occs_binary.astype(jnp.uint8)
    tbl = lax.bitcast_convert_type(occ8.reshape(_TBL_WORDS, 4), jnp.int32)
    out = _occupied(posf, tbl)
    return out.astype(jnp.bool_)


# speedup vs baseline: 16.0348x; 16.0348x over previous
import functools

import jax
import jax.numpy as jnp
import numpy as np
from jax import lax
from jax.experimental import pallas as pl
from jax.experimental.pallas import tpu as pltpu
from jax.experimental.pallas import tpu_sc as plsc

_N = 4194304
_NC, _NS = 2, 16
_NW = _NC * _NS
_PPW = _N // _NW
_C = 4096
_NCHUNK = _PPW // _C
_R = _C // 128
_GPR = 8

_F_HALF = np.float32(0.5)
_F_ONE = np.float32(1.0)
_F_RES = np.float32(128.0)
_F_TOP = np.float32(1.0 - 1e-5)
_SIGN = np.int32(-2147483648)


def _sc_body(x_hbm, y_hbm, z_hbm, tbl_hbm, out_hbm,
             xv, yv, zv, widxv, vidxv, wordsv, outv, sem):
    wid = lax.axis_index("s") * _NC + lax.axis_index("c")
    base = wid * _PPW

    def chunk(c, _):
        start = base + c * _C
        pltpu.sync_copy(x_hbm.at[pl.ds(start, _C)], xv)
        pltpu.sync_copy(y_hbm.at[pl.ds(start, _C)], yv)
        pltpu.sync_copy(z_hbm.at[pl.ds(start, _C)], zv)

        def row_compute(j, _):
            for gg in range(_GPR):
                g = j * _GPR + gg
                sl = pl.ds(g * 16, 16)
                px = xv[sl]
                py = yv[sl]
                pz = zv[sl]
                tx = (px + _F_ONE) * _F_HALF - _F_HALF
                ty = (py + _F_ONE) * _F_HALF - _F_HALF
                tz = (pz + _F_ONE) * _F_HALF - _F_HALF
                m = jnp.maximum(jnp.maximum(jnp.abs(tx), jnp.abs(ty)),
                                jnp.abs(tz))
                ebits = plsc.bitcast(m, jnp.int32) >> 23
                mip_raw = jnp.where(m > 0.0, ebits - 125, 1)
                valid = mip_raw < 4
                mip = jnp.minimum(jnp.maximum(mip_raw, 0), 3)
                scale = plsc.bitcast((127 - mip) << 23, jnp.float32)
                vx = jnp.minimum(jnp.maximum(tx * scale + _F_HALF, 0.0), _F_TOP)
                vy = jnp.minimum(jnp.maximum(ty * scale + _F_HALF, 0.0), _F_TOP)
                vz = jnp.minimum(jnp.maximum(tz * scale + _F_HALF, 0.0), _F_TOP)
                xi = (vx * _F_RES).astype(jnp.int32)
                yi = (vy * _F_RES).astype(jnp.int32)
                zi = (vz * _F_RES).astype(jnp.int32)
                idx = xi * 16384 + yi * 128 + zi + (mip << 21)
                widxv[sl] = idx
                vidxv[sl] = jnp.where(valid, idx, idx | _SIGN)
            pltpu.async_copy(
                tbl_hbm.at[widxv.at[pl.ds(j * 128, 128)]],
                wordsv.at[pl.ds(j * 128, 128)], sem)
            return ()

        lax.fori_loop(0, _R, row_compute, (), unroll=False)

        def row_drain(j, _):
            pltpu.make_async_copy(
                tbl_hbm.at[widxv.at[pl.ds(j * 128, 128)]],
                wordsv.at[pl.ds(j * 128, 128)], sem).wait()
            return ()

        lax.fori_loop(0, _R, row_drain, (), unroll=False)

        def row_extract(j, _):
            for gg in range(_GPR):
                g = j * _GPR + gg
                sl = pl.ds(g * 16, 16)
                w = wordsv[sl]
                outv[sl] = jnp.where(vidxv[sl] >= 0, w, 0)
            return ()

        lax.fori_loop(0, _R, row_extract, (), unroll=False)
        pltpu.sync_copy(outv, out_hbm.at[pl.ds(start, _C)])
        return ()

    lax.fori_loop(0, _NCHUNK, chunk, (), unroll=False)


@jax.jit
def _occupied(x, y, z, tbl):
    mesh = plsc.VectorSubcoreMesh(
        core_axis_name="c", subcore_axis_name="s",
        num_cores=_NC, num_subcores=_NS)
    f = pl.kernel(
        _sc_body,
        out_type=jax.ShapeDtypeStruct((_N,), jnp.int32),
        mesh=mesh,
        compiler_params=pltpu.CompilerParams(needs_layout_passes=False),
        scratch_types=[
            pltpu.VMEM((_C,), jnp.float32),
            pltpu.VMEM((_C,), jnp.float32),
            pltpu.VMEM((_C,), jnp.float32),
            pltpu.VMEM((_C,), jnp.int32),
            pltpu.VMEM((_C,), jnp.int32),
            pltpu.VMEM((_C,), jnp.int32),
            pltpu.VMEM((_C,), jnp.int32),
            pltpu.SemaphoreType.DMA,
        ],
    )
    return f(x, y, z, tbl)


def kernel(pos, occs_binary, aabbs):
    x = pos[:, 0]
    y = pos[:, 1]
    z = pos[:, 2]
    tbl = occs_binary.astype(jnp.int32)
    out = _occupied(x, y, z, tbl)
    return out.astype(jnp.bool_)


# speedup vs baseline: 21.0857x; 1.3150x over previous
import functools

import jax
import jax.numpy as jnp
import numpy as np
from jax import lax
from jax.experimental import pallas as pl
from jax.experimental.pallas import tpu as pltpu
from jax.experimental.pallas import tpu_sc as plsc

_N = 4194304
_TBL = 8388608
_PAD = 128
_NC, _NS = 2, 16
_NW = _NC * _NS
_PPW = _N // _NW
_C = 4096
_NCHUNK = _PPW // _C
_NPAIR = _NCHUNK // 2
_R = _C // 128
_GPR = 8

_F_HALF = np.float32(0.5)
_F_ONE = np.float32(1.0)
_F_RES = np.float32(128.0)
_F_TOP = np.float32(1.0 - 1e-5)
_PBASE = np.int32(_TBL)


def _sc_body(x_hbm, y_hbm, z_hbm, tbl_hbm, out_hbm,
             xvA, yvA, zvA, wvA, ovA,
             xvB, yvB, zvB, wvB, ovB,
             semA, semB, sxA, sxB):
    wid = lax.axis_index("s") * _NC + lax.axis_index("c")
    base = wid * _PPW

    def start_xyz(c, xv, yv, zv, sx):
        s = base + c * _C
        pltpu.async_copy(x_hbm.at[pl.ds(s, _C)], xv, sx)
        pltpu.async_copy(y_hbm.at[pl.ds(s, _C)], yv, sx)
        pltpu.async_copy(z_hbm.at[pl.ds(s, _C)], zv, sx)

    def wait_xyz(xv, yv, zv, sx):
        pltpu.make_async_copy(x_hbm.at[pl.ds(0, _C)], xv, sx).wait()
        pltpu.make_async_copy(y_hbm.at[pl.ds(0, _C)], yv, sx).wait()
        pltpu.make_async_copy(z_hbm.at[pl.ds(0, _C)], zv, sx).wait()

    def compute_fire(xv, yv, zv, wv, ov, sem):
        def row(j, _):
            for gg in range(_GPR):
                g = j * _GPR + gg
                sl = pl.ds(g * 16, 16)
                px = xv[sl]
                py = yv[sl]
                pz = zv[sl]
                tx = (px + _F_ONE) * _F_HALF - _F_HALF
                ty = (py + _F_ONE) * _F_HALF - _F_HALF
                tz = (pz + _F_ONE) * _F_HALF - _F_HALF
                m = jnp.maximum(jnp.maximum(jnp.abs(tx), jnp.abs(ty)),
                                jnp.abs(tz))
                ebits = plsc.bitcast(m, jnp.int32) >> 23
                mip_raw = jnp.where(m > 0.0, ebits - 125, 1)
                valid = mip_raw < 4
                mip = jnp.minimum(jnp.maximum(mip_raw, 0), 3)
                scale = plsc.bitcast((127 - mip) << 23, jnp.float32)
                vx = jnp.minimum(jnp.maximum(tx * scale + _F_HALF, 0.0), _F_TOP)
                vy = jnp.minimum(jnp.maximum(ty * scale + _F_HALF, 0.0), _F_TOP)
                vz = jnp.minimum(jnp.maximum(tz * scale + _F_HALF, 0.0), _F_TOP)
                xi = (vx * _F_RES).astype(jnp.int32)
                yi = (vy * _F_RES).astype(jnp.int32)
                zi = (vz * _F_RES).astype(jnp.int32)
                idx = xi * 16384 + yi * 128 + zi + (mip << 21)
                wv[sl] = jnp.where(valid, idx, _PBASE + (idx & 127))
            pltpu.async_copy(
                tbl_hbm.at[wv.at[pl.ds(j * 128, 128)]],
                ov.at[pl.ds(j * 128, 128)], sem)
            return ()

        lax.fori_loop(0, _R, row, (), unroll=False)

    def drain_store(c, wv, ov, sem):
        def row(j, _):
            pltpu.make_async_copy(
                tbl_hbm.at[wv.at[pl.ds(j * 128, 128)]],
                ov.at[pl.ds(j * 128, 128)], sem).wait()
            return ()

        lax.fori_loop(0, _R, row, (), unroll=False)
        pltpu.sync_copy(ov, out_hbm.at[pl.ds(base + c * _C, _C)])

    start_xyz(0, xvA, yvA, zvA, sxA)
    start_xyz(1, xvB, yvB, zvB, sxB)
    wait_xyz(xvA, yvA, zvA, sxA)
    compute_fire(xvA, yvA, zvA, wvA, ovA, semA)

    def pair(c2, _):
        c0 = c2 * 2
        wait_xyz(xvB, yvB, zvB, sxB)
        compute_fire(xvB, yvB, zvB, wvB, ovB, semB)

        @pl.when(c2 < _NPAIR - 1)
        def _():
            start_xyz(c0 + 2, xvA, yvA, zvA, sxA)

        drain_store(c0, wvA, ovA, semA)

        @pl.when(c2 < _NPAIR - 1)
        def _():
            wait_xyz(xvA, yvA, zvA, sxA)
            compute_fire(xvA, yvA, zvA, wvA, ovA, semA)
            start_xyz(c0 + 3, xvB, yvB, zvB, sxB)

        drain_store(c0 + 1, wvB, ovB, semB)
        return ()

    lax.fori_loop(0, _NPAIR, pair, (), unroll=False)


@jax.jit
def _occupied(x, y, z, tbl):
    mesh = plsc.VectorSubcoreMesh(
        core_axis_name="c", subcore_axis_name="s",
        num_cores=_NC, num_subcores=_NS)
    buf = [
        pltpu.VMEM((_C,), jnp.float32),
        pltpu.VMEM((_C,), jnp.float32),
        pltpu.VMEM((_C,), jnp.float32),
        pltpu.VMEM((_C,), jnp.int32),
        pltpu.VMEM((_C,), jnp.int32),
    ]
    f = pl.kernel(
        _sc_body,
        out_type=jax.ShapeDtypeStruct((_N,), jnp.int32),
        mesh=mesh,
        compiler_params=pltpu.CompilerParams(needs_layout_passes=False),
        scratch_types=buf + buf + [
            pltpu.SemaphoreType.DMA,
            pltpu.SemaphoreType.DMA,
            pltpu.SemaphoreType.DMA,
            pltpu.SemaphoreType.DMA,
        ],
    )
    return f(x, y, z, tbl)


def kernel(pos, occs_binary, aabbs):
    x = pos[:, 0]
    y = pos[:, 1]
    z = pos[:, 2]
    tbl = jnp.pad(occs_binary.astype(jnp.int32), (0, _PAD))
    out = _occupied(x, y, z, tbl)
    return out.astype(jnp.bool_)
